# baseline (speedup 1.0000x reference)
import functools

import jax
import jax.numpy as jnp
from jax import lax
from jax.experimental import pallas as pl
from jax.experimental.pallas import tpu as pltpu
from jax.experimental.pallas import tpu_sc as plsc

LANES = 16
NW = 32
CR = 8
NBUF = 3


def _vgather(x, idx):
    return lax.gather(
        x,
        idx[:, None],
        dimension_numbers=lax.GatherDimensionNumbers(
            offset_dims=(), collapsed_slice_dims=(0,), start_index_map=(0,)
        ),
        slice_sizes=(1,),
        mode=lax.GatherScatterMode.PROMISE_IN_BOUNDS,
    )


def _align_vreg(x, o16, base16, cneg, cpos, p1, p2, p4):
    bits = lax.bitcast_convert_type(x, jnp.int32)
    km = (bits & jnp.int32(-8)) ^ jnp.where(bits < 0, cneg, cpos)
    km = jnp.minimum(km, _vgather(km, p1))
    km = jnp.minimum(km, _vgather(km, p2))
    km = jnp.minimum(km, _vgather(km, p4))
    d = km & 7
    idx = base16 | ((o16 + d) & 7)
    return _vgather(x, idx)


def _tc_align_block(x_ref, o_ref):
    x = x_ref[...]
    shape = x.shape
    o = lax.broadcasted_iota(jnp.int32, shape, 1) & 7
    cneg = jnp.int32(-(2**31)) | o
    cpos = jnp.int32(-8) ^ o
    bits = lax.bitcast_convert_type(x, jnp.int32)
    km = (bits & jnp.int32(-8)) ^ jnp.where(bits < 0, cneg, cpos)

    def roll_l(y, s):
        return jnp.concatenate([y[:, s:], y[:, :s]], axis=1)

    def roll_r(y, s):
        return jnp.concatenate([y[:, -s:], y[:, :-s]], axis=1)

    for s in (1, 2, 4):
        paired = jnp.where((o & s) == 0, roll_l(km, s), roll_r(km, s))
        km = jnp.minimum(km, paired)
    d = km & 7
    y = x
    for s in (1, 2, 4):
        rolled = jnp.where(o < 8 - s, roll_l(y, s), roll_r(y, 8 - s))
        y = jnp.where((d & s) != 0, rolled, y)
    o_ref[...] = y


def kernel(input):
    N, C = input.shape
    n_sc = 8192
    rows_w = n_sc // NW
    n_chunks = rows_w // CR
    vregs_row = C // LANES

    mesh = plsc.VectorSubcoreMesh(core_axis_name="c", subcore_axis_name="s")

    @functools.partial(
        pl.kernel,
        mesh=mesh,
        out_type=jax.ShapeDtypeStruct((n_sc, C), jnp.float32),
        scratch_types=[
            pltpu.VMEM((NBUF, CR, C), jnp.float32),
            pltpu.SemaphoreType.DMA,
            pltpu.SemaphoreType.DMA,
            pltpu.SemaphoreType.DMA,
            pltpu.SemaphoreType.DMA,
            pltpu.SemaphoreType.DMA,
            pltpu.SemaphoreType.DMA,
        ],
        compiler_params=pltpu.CompilerParams(use_tc_tiling_on_sc=True),
    )
    def run(x_hbm, out_hbm, bufs, si0, si1, si2, so0, so1, so2):
        wid = lax.axis_index("s") * 2 + lax.axis_index("c")
        row0 = wid * rows_w
        iota = lax.iota(jnp.int32, LANES)
        o16 = iota & 7
        base16 = iota & jnp.int32(-8)
        p1 = iota ^ 1
        p2 = iota ^ 2
        p4 = iota ^ 4
        cneg = jnp.int32(-(2**31)) | o16
        cpos = jnp.int32(-8) ^ o16
        sem_in = (si0, si1, si2)
        sem_out = (so0, so1, so2)

        def in_slice(ci):
            return x_hbm.at[pl.ds(row0 + ci * CR, CR), :]

        def out_slice(ci):
            return out_hbm.at[pl.ds(row0 + ci * CR, CR), :]

        for b in range(2):
            pltpu.async_copy(in_slice(b), bufs.at[b], sem_in[b])

        def do_chunk(ci, b):
            buf = bufs.at[b]
            pltpu.make_async_copy(in_slice(ci), buf, sem_in[b]).wait()

            for r in range(CR):

                @plsc.parallel_loop(0, vregs_row, unroll=16)
                def body(v):
                    x = buf[r, pl.ds(v * LANES, LANES)]
                    buf[r, pl.ds(v * LANES, LANES)] = _align_vreg(
                        x, o16, base16, cneg, cpos, p1, p2, p4
                    )

            pltpu.async_copy(buf, out_slice(ci), sem_out[b])

            b2 = (b + 2) % NBUF

            @pl.when(ci + 2 < n_chunks)
            def _():
                @pl.when(ci >= 1)
                def _():
                    pltpu.make_async_copy(
                        bufs.at[b2], out_slice(ci - 1), sem_out[b2]
                    ).wait()

                pltpu.async_copy(in_slice(ci + 2), bufs.at[b2], sem_in[b2])

        def outer(t, _):
            for b in range(NBUF):
                do_chunk(t * NBUF + b, b)
            return 0

        lax.fori_loop(0, n_chunks // NBUF, outer, 0)
        for ci in range((n_chunks // NBUF) * NBUF, n_chunks):
            do_chunk(ci, ci % NBUF)

        for k in range(NBUF):
            ci = n_chunks - NBUF + k
            pltpu.make_async_copy(
                bufs.at[ci % NBUF], out_slice(ci), sem_out[ci % NBUF]
            ).wait()

    sc_out = run(input)

    n_tc = N - n_sc
    if n_tc == 0:
        return sc_out
    br = 256
    tc_out = pl.pallas_call(
        _tc_align_block,
        grid=(n_tc // br,),
        in_specs=[pl.BlockSpec((br, C), lambda i: (i + n_sc // br, 0))],
        out_specs=pl.BlockSpec((br, C), lambda i: (i, 0)),
        out_shape=jax.ShapeDtypeStruct((n_tc, C), jnp.float32),
    )(input)
    return jnp.concatenate([sc_out, tc_out], axis=0)


# speedup vs baseline: 1.5769x; 1.5769x over previous
import functools

import jax
import jax.numpy as jnp
from jax import lax
from jax.experimental import pallas as pl
from jax.experimental.pallas import tpu as pltpu
from jax.experimental.pallas import tpu_sc as plsc

LANES = 16
NW = 32
CR = 8
NBUF = 3


def _vgather(x, idx):
    return lax.gather(
        x,
        idx[:, None],
        dimension_numbers=lax.GatherDimensionNumbers(
            offset_dims=(), collapsed_slice_dims=(0,), start_index_map=(0,)
        ),
        slice_sizes=(1,),
        mode=lax.GatherScatterMode.PROMISE_IN_BOUNDS,
    )


def _align_vreg(x, o16, base16, cneg, cpos, c7f, p1, p2, p4):
    bits = lax.bitcast_convert_type(x, jnp.uint32)
    km = (bits & jnp.uint32(0xFFFFFFF0)) ^ jnp.where(bits > c7f, cneg, cpos)
    km = jnp.minimum(km, _vgather(km, p1))
    km = jnp.minimum(km, _vgather(km, p2))
    km = jnp.minimum(km, _vgather(km, p4))
    kmi = lax.bitcast_convert_type(km, jnp.int32)
    idx = base16 | ((o16 + kmi) & 7)
    return _vgather(x, idx)


def kernel(input):
    N, C = input.shape
    n_sc = N
    rows_w = n_sc // NW
    n_chunks = rows_w // CR
    vregs_row = C // LANES

    mesh = plsc.VectorSubcoreMesh(core_axis_name="c", subcore_axis_name="s")

    @functools.partial(
        pl.kernel,
        mesh=mesh,
        out_type=jax.ShapeDtypeStruct((n_sc, C), jnp.float32),
        scratch_types=[
            pltpu.VMEM((NBUF, CR, C), jnp.float32),
            pltpu.SemaphoreType.DMA,
            pltpu.SemaphoreType.DMA,
            pltpu.SemaphoreType.DMA,
            pltpu.SemaphoreType.DMA,
            pltpu.SemaphoreType.DMA,
            pltpu.SemaphoreType.DMA,
        ],
        compiler_params=pltpu.CompilerParams(use_tc_tiling_on_sc=True),
    )
    def run(x_hbm, out_hbm, bufs, si0, si1, si2, so0, so1, so2):
        wid = lax.axis_index("s") * 2 + lax.axis_index("c")
        row0 = wid * rows_w
        iota = lax.iota(jnp.int32, LANES)
        o16 = iota & 7
        base16 = iota & jnp.int32(-8)
        p1 = iota ^ 1
        p2 = iota ^ 2
        p4 = iota ^ 4
        iota_u = lax.bitcast_convert_type(iota, jnp.uint32)
        cneg = iota_u
        cpos = jnp.uint32(0x7FFFFFF0) ^ iota_u
        c7f = jnp.uint32(0x7FFFFFFF)
        sem_in = (si0, si1, si2)
        sem_out = (so0, so1, so2)

        def in_slice(ci):
            return x_hbm.at[pl.ds(row0 + ci * CR, CR), :]

        def out_slice(ci):
            return out_hbm.at[pl.ds(row0 + ci * CR, CR), :]

        for b in range(2):
            pltpu.async_copy(in_slice(b), bufs.at[b], sem_in[b])

        def do_chunk(ci, b):
            buf = bufs.at[b]
            pltpu.make_async_copy(in_slice(ci), buf, sem_in[b]).wait()

            for r in range(CR):

                @plsc.parallel_loop(0, vregs_row, unroll=16)
                def body(v):
                    x = buf[r, pl.ds(v * LANES, LANES)]
                    buf[r, pl.ds(v * LANES, LANES)] = _align_vreg(
                        x, o16, base16, cneg, cpos, c7f, p1, p2, p4
                    )

            pltpu.async_copy(buf, out_slice(ci), sem_out[b])

            b2 = (b + 2) % NBUF

            @pl.when(ci + 2 < n_chunks)
            def _():
                @pl.when(ci >= 1)
                def _():
                    pltpu.make_async_copy(
                        bufs.at[b2], out_slice(ci - 1), sem_out[b2]
                    ).wait()

                pltpu.async_copy(in_slice(ci + 2), bufs.at[b2], sem_in[b2])

        def outer(t, _):
            for b in range(NBUF):
                do_chunk(t * NBUF + b, b)
            return 0

        lax.fori_loop(0, n_chunks // NBUF, outer, 0)
        for ci in range((n_chunks // NBUF) * NBUF, n_chunks):
            do_chunk(ci, ci % NBUF)

        for k in range(NBUF):
            ci = n_chunks - NBUF + k
            pltpu.make_async_copy(
                bufs.at[ci % NBUF], out_slice(ci), sem_out[ci % NBUF]
            ).wait()

    return run(input)
